# Initial kernel scaffold; baseline (speedup 1.0000x reference)
import jax
import jax.numpy as jnp
from jax.experimental import pallas as pl


def kernel(embedding, edge_index):
    raise NotImplementedError("write your pallas kernel here")


# speedup vs baseline: 1.4629x; 1.4629x over previous
import functools

import jax
import jax.numpy as jnp
from jax import lax
from jax.experimental import pallas as pl
from jax.experimental.pallas import tpu as pltpu
from jax.experimental.pallas import tpu_sc as plsc

_LAMBDA = 0.2
_E = 160000
_D = 256
_NW = 32
_EPW = 5120
_K = 128
_NCHUNK = _EPW // _K
_E_PAD = _NW * _EPW


def _rsqrt(p):
    i = lax.bitcast_convert_type(p, jnp.int32)
    i = jnp.int32(0x5F3759DF) - (i >> 1)
    y = lax.bitcast_convert_type(i, jnp.float32)
    y = y * (1.5 - 0.5 * p * y * y)
    y = y * (1.5 - 0.5 * p * y * y)
    y = y * (1.5 - 0.5 * p * y * y)
    return y


def _build():
    mesh = plsc.VectorSubcoreMesh(core_axis_name="c", subcore_axis_name="s")

    @functools.partial(
        pl.kernel,
        out_type=jax.ShapeDtypeStruct((_NW, 32), jnp.float32),
        mesh=mesh,
        compiler_params=pltpu.CompilerParams(needs_layout_passes=False),
        scratch_types=[
            pltpu.VMEM((_K,), jnp.int32),
            pltpu.VMEM((_K,), jnp.int32),
            pltpu.VMEM((_K, _D), jnp.float32),
            pltpu.VMEM((_K, _D), jnp.float32),
            pltpu.VMEM((32,), jnp.float32),
            pltpu.SemaphoreType.DMA,
            pltpu.SemaphoreType.DMA,
        ],
    )
    def k(table, sidx_hbm, tidx_hbm, out,
          sidx, tidx, srows, trows, outbuf, sem1, sem2):
        wid = lax.axis_index("s") * 2 + lax.axis_index("c")
        base_w = wid * _EPW

        def chunk_body(j, carry):
            acc_cos, acc_sq = carry
            base = base_w + j * _K
            pltpu.sync_copy(sidx_hbm.at[pl.ds(base, _K)], sidx)
            pltpu.sync_copy(tidx_hbm.at[pl.ds(base, _K)], tidx)
            cp1 = pltpu.async_copy(table.at[sidx], srows, sem1)
            cp2 = pltpu.async_copy(table.at[tidx], trows, sem2)
            cp1.wait()
            cp2.wait()

            def edge_body(i, carry2):
                a_cos, a_sq = carry2
                sv = srows[i, pl.ds(0, 16)]
                tv = trows[i, pl.ds(0, 16)]
                d = sv * tv
                ns = sv * sv
                nt = tv * tv
                for c in range(1, 16):
                    sv = srows[i, pl.ds(c * 16, 16)]
                    tv = trows[i, pl.ds(c * 16, 16)]
                    d = d + sv * tv
                    ns = ns + sv * sv
                    nt = nt + tv * tv
                dsc = jnp.sum(d)
                nssc = jnp.sum(ns)
                ntsc = jnp.sum(nt)
                p = jnp.maximum(nssc, 1e-16) * jnp.maximum(ntsc, 1e-16)
                cos = dsc * _rsqrt(p)
                valid = (base + i) < _E
                a_cos = a_cos + jnp.where(valid, 1.0 - cos, 0.0)
                a_sq = a_sq + jnp.where(valid, nssc + ntsc - 2.0 * dsc, 0.0)
                return (a_cos, a_sq)

            return lax.fori_loop(0, _K, edge_body, (acc_cos, acc_sq))

        acc_cos, acc_sq = lax.fori_loop(
            0, _NCHUNK, chunk_body, (jnp.float32(0.0), jnp.float32(0.0)))
        lanes = lax.iota(jnp.int32, 16)
        first = lanes < 1
        outbuf[pl.ds(0, 16)] = jnp.where(
            first, acc_cos * (1.0 / _E), 0.0)
        outbuf[pl.ds(16, 16)] = jnp.where(
            first, acc_sq * (_LAMBDA / (_E * _D)), 0.0)
        pltpu.sync_copy(outbuf, out.at[wid])

    return k


_sc_kernel = _build()


def kernel(embedding, edge_index):
    ei = edge_index.astype(jnp.int32)
    pad = _E_PAD - _E
    src = jnp.concatenate([ei[0], jnp.zeros((pad,), jnp.int32)])
    tgt = jnp.concatenate([ei[1], jnp.zeros((pad,), jnp.int32)])
    parts = _sc_kernel(embedding, src, tgt)
    return jnp.sum(parts)


# speedup vs baseline: 1.7451x; 1.1929x over previous
import functools

import jax
import jax.numpy as jnp
from jax import lax
from jax.experimental import pallas as pl
from jax.experimental.pallas import tpu as pltpu
from jax.experimental.pallas import tpu_sc as plsc

_LAMBDA = 0.2
_E = 160000
_D = 256
_NW = 32
_EPW = 5120
_K = 64
_NPAIR = _EPW // (2 * _K)
_NCHUNK = _EPW // _K
_E_PAD = _NW * _EPW


def _rsqrt(p):
    i = lax.bitcast_convert_type(p, jnp.int32)
    i = jnp.int32(0x5F3759DF) - (i >> 1)
    y = lax.bitcast_convert_type(i, jnp.float32)
    y = y * (1.5 - 0.5 * p * y * y)
    y = y * (1.5 - 0.5 * p * y * y)
    y = y * (1.5 - 0.5 * p * y * y)
    return y


def _build():
    mesh = plsc.VectorSubcoreMesh(core_axis_name="c", subcore_axis_name="s")

    @functools.partial(
        pl.kernel,
        out_type=jax.ShapeDtypeStruct((_NW, 32), jnp.float32),
        mesh=mesh,
        compiler_params=pltpu.CompilerParams(needs_layout_passes=False),
        scratch_types=[
            pltpu.VMEM((_K,), jnp.int32),
            pltpu.VMEM((_K,), jnp.int32),
            pltpu.VMEM((_K, _D), jnp.float32),
            pltpu.VMEM((_K, _D), jnp.float32),
            pltpu.VMEM((_K,), jnp.int32),
            pltpu.VMEM((_K,), jnp.int32),
            pltpu.VMEM((_K, _D), jnp.float32),
            pltpu.VMEM((_K, _D), jnp.float32),
            pltpu.VMEM((32,), jnp.float32),
            pltpu.SemaphoreType.DMA,
            pltpu.SemaphoreType.DMA,
            pltpu.SemaphoreType.DMA,
            pltpu.SemaphoreType.DMA,
        ],
    )
    def k(table, sidx_hbm, tidx_hbm, out,
          sidx0, tidx0, srows0, trows0,
          sidx1, tidx1, srows1, trows1,
          outbuf, ss0, st0, ss1, st1):
        wid = lax.axis_index("s") * 2 + lax.axis_index("c")
        base_w = wid * _EPW
        bufs = ((sidx0, tidx0, srows0, trows0, ss0, st0),
                (sidx1, tidx1, srows1, trows1, ss1, st1))

        def fire(jb, b):
            sidx, tidx, srows, trows, ss, st = bufs[b]
            base = base_w + jb * _K
            pltpu.sync_copy(sidx_hbm.at[pl.ds(base, _K)], sidx)
            pltpu.sync_copy(tidx_hbm.at[pl.ds(base, _K)], tidx)
            pltpu.async_copy(table.at[sidx], srows, ss)
            pltpu.async_copy(table.at[tidx], trows, st)

        def drain(b):
            sidx, tidx, srows, trows, ss, st = bufs[b]
            pltpu.make_async_copy(table.at[sidx], srows, ss).wait()
            pltpu.make_async_copy(table.at[tidx], trows, st).wait()

        def compute(jb, b, acc):
            _, _, srows, trows, _, _ = bufs[b]
            base = base_w + jb * _K

            def edge_body(i, carry):
                a_cos, a_sq = carry
                sv = srows[i, pl.ds(0, 16)]
                tv = trows[i, pl.ds(0, 16)]
                d = sv * tv
                ns = sv * sv
                nt = tv * tv
                for c in range(1, 16):
                    sv = srows[i, pl.ds(c * 16, 16)]
                    tv = trows[i, pl.ds(c * 16, 16)]
                    d = d + sv * tv
                    ns = ns + sv * sv
                    nt = nt + tv * tv
                dsc = jnp.sum(d)
                nssc = jnp.sum(ns)
                ntsc = jnp.sum(nt)
                p = jnp.maximum(nssc, 1e-16) * jnp.maximum(ntsc, 1e-16)
                cos = dsc * _rsqrt(p)
                valid = (base + i) < _E
                a_cos = a_cos + jnp.where(valid, 1.0 - cos, 0.0)
                a_sq = a_sq + jnp.where(valid, nssc + ntsc - 2.0 * dsc, 0.0)
                return (a_cos, a_sq)

            return plsc.parallel_loop(0, _K, 1, unroll=4, carry=acc)(edge_body)

        fire(0, 0)

        def pair_body(j2, acc):
            a = 2 * j2
            fire(a + 1, 1)
            drain(0)
            acc = compute(a, 0, acc)

            @pl.when(j2 < _NPAIR - 1)
            def _():
                fire(a + 2, 0)

            drain(1)
            acc = compute(a + 1, 1, acc)
            return acc

        acc_cos, acc_sq = lax.fori_loop(
            0, _NPAIR, pair_body, (jnp.float32(0.0), jnp.float32(0.0)))
        lanes = lax.iota(jnp.int32, 16)
        first = lanes < 1
        outbuf[pl.ds(0, 16)] = jnp.where(
            first, acc_cos * (1.0 / _E), 0.0)
        outbuf[pl.ds(16, 16)] = jnp.where(
            first, acc_sq * (_LAMBDA / (_E * _D)), 0.0)
        pltpu.sync_copy(outbuf, out.at[wid])

    return k


_sc_kernel = _build()


def kernel(embedding, edge_index):
    ei = edge_index.astype(jnp.int32)
    pad = _E_PAD - _E
    src = jnp.concatenate([ei[0], jnp.zeros((pad,), jnp.int32)])
    tgt = jnp.concatenate([ei[1], jnp.zeros((pad,), jnp.int32)])
    parts = _sc_kernel(embedding, src, tgt)
    return jnp.sum(parts)


# speedup vs baseline: 1.7929x; 1.0274x over previous
import functools

import jax
import jax.numpy as jnp
from jax import lax
from jax.experimental import pallas as pl
from jax.experimental.pallas import tpu as pltpu
from jax.experimental.pallas import tpu_sc as plsc

_LAMBDA = 0.2
_E = 160000
_D = 256
_NW = 32
_EPW = 5120
_K = 128
_NPAIR = _EPW // (2 * _K)
_NCHUNK = _EPW // _K
_E_PAD = _NW * _EPW


def _rsqrt(p):
    i = lax.bitcast_convert_type(p, jnp.int32)
    i = jnp.int32(0x5F3759DF) - (i >> 1)
    y = lax.bitcast_convert_type(i, jnp.float32)
    y = y * (1.5 - 0.5 * p * y * y)
    y = y * (1.5 - 0.5 * p * y * y)
    y = y * (1.5 - 0.5 * p * y * y)
    return y


def _build():
    mesh = plsc.VectorSubcoreMesh(core_axis_name="c", subcore_axis_name="s")

    @functools.partial(
        pl.kernel,
        out_type=jax.ShapeDtypeStruct((_NW, 32), jnp.float32),
        mesh=mesh,
        compiler_params=pltpu.CompilerParams(needs_layout_passes=False),
        scratch_types=[
            pltpu.VMEM((_K,), jnp.int32),
            pltpu.VMEM((_K,), jnp.int32),
            pltpu.VMEM((_K, _D // 2), jnp.float32),
            pltpu.VMEM((_K, _D // 2), jnp.float32),
            pltpu.VMEM((_K,), jnp.int32),
            pltpu.VMEM((_K,), jnp.int32),
            pltpu.VMEM((_K, _D // 2), jnp.float32),
            pltpu.VMEM((_K, _D // 2), jnp.float32),
            pltpu.VMEM((32,), jnp.float32),
            pltpu.SemaphoreType.DMA,
            pltpu.SemaphoreType.DMA,
            pltpu.SemaphoreType.DMA,
            pltpu.SemaphoreType.DMA,
        ],
    )
    def k(table, sidx_hbm, tidx_hbm, out,
          sidx0, tidx0, srows0, trows0,
          sidx1, tidx1, srows1, trows1,
          outbuf, ss0, st0, ss1, st1):
        wid = lax.axis_index("s") * 2 + lax.axis_index("c")
        base_w = wid * _EPW
        bufs = ((sidx0, tidx0, srows0, trows0, ss0, st0),
                (sidx1, tidx1, srows1, trows1, ss1, st1))

        def fire(jb, b):
            sidx, tidx, srows, trows, ss, st = bufs[b]
            base = base_w + jb * _K
            pltpu.sync_copy(sidx_hbm.at[pl.ds(base, _K)], sidx)
            pltpu.sync_copy(tidx_hbm.at[pl.ds(base, _K)], tidx)
            pltpu.async_copy(table.at[sidx], srows, ss)
            pltpu.async_copy(table.at[tidx], trows, st)

        def drain(b):
            sidx, tidx, srows, trows, ss, st = bufs[b]
            pltpu.make_async_copy(table.at[sidx], srows, ss).wait()
            pltpu.make_async_copy(table.at[tidx], trows, st).wait()

        def compute(jb, b, acc):
            _, _, srows, trows, _, _ = bufs[b]
            base = base_w + jb * _K

            def edge_body(i, carry):
                a_cos, a_sq = carry
                d = ns = nt = None
                for c in range(8):
                    sa, sb = plsc.unpack(
                        plsc.bitcast(srows[i, pl.ds(c * 16, 16)], jnp.bfloat16),
                        format=plsc.PackFormat.INTERLEAVED,
                        preferred_element_type=jnp.float32)
                    ta, tb = plsc.unpack(
                        plsc.bitcast(trows[i, pl.ds(c * 16, 16)], jnp.bfloat16),
                        format=plsc.PackFormat.INTERLEAVED,
                        preferred_element_type=jnp.float32)
                    if d is None:
                        d = sa * ta + sb * tb
                        ns = sa * sa + sb * sb
                        nt = ta * ta + tb * tb
                    else:
                        d = d + sa * ta + sb * tb
                        ns = ns + sa * sa + sb * sb
                        nt = nt + ta * ta + tb * tb
                dsc = jnp.sum(d)
                nssc = jnp.sum(ns)
                ntsc = jnp.sum(nt)
                p = jnp.maximum(nssc, 1e-16) * jnp.maximum(ntsc, 1e-16)
                cos = dsc * _rsqrt(p)
                valid = (base + i) < _E
                a_cos = a_cos + jnp.where(valid, 1.0 - cos, 0.0)
                a_sq = a_sq + jnp.where(valid, nssc + ntsc - 2.0 * dsc, 0.0)
                return (a_cos, a_sq)

            return plsc.parallel_loop(0, _K, 1, unroll=4, carry=acc)(edge_body)

        fire(0, 0)

        def pair_body(j2, acc):
            a = 2 * j2
            fire(a + 1, 1)
            drain(0)
            acc = compute(a, 0, acc)

            @pl.when(j2 < _NPAIR - 1)
            def _():
                fire(a + 2, 0)

            drain(1)
            acc = compute(a + 1, 1, acc)
            return acc

        acc_cos, acc_sq = lax.fori_loop(
            0, _NPAIR, pair_body, (jnp.float32(0.0), jnp.float32(0.0)))
        lanes = lax.iota(jnp.int32, 16)
        first = lanes < 1
        outbuf[pl.ds(0, 16)] = jnp.where(
            first, acc_cos * (1.0 / _E), 0.0)
        outbuf[pl.ds(16, 16)] = jnp.where(
            first, acc_sq * (_LAMBDA / (_E * _D)), 0.0)
        pltpu.sync_copy(outbuf, out.at[wid])

    return k


_sc_kernel = _build()


def kernel(embedding, edge_index):
    ei = edge_index.astype(jnp.int32)
    pad = _E_PAD - _E
    src = jnp.concatenate([ei[0], jnp.zeros((pad,), jnp.int32)])
    tgt = jnp.concatenate([ei[1], jnp.zeros((pad,), jnp.int32)])
    table = lax.bitcast_convert_type(
        embedding.astype(jnp.bfloat16).reshape(-1, _D // 2, 2), jnp.float32)
    parts = _sc_kernel(table, src, tgt)
    return jnp.sum(parts)


# speedup vs baseline: 2.0944x; 1.1682x over previous
import functools

import jax
import jax.numpy as jnp
from jax import lax
from jax.experimental import pallas as pl
from jax.experimental.pallas import tpu as pltpu
from jax.experimental.pallas import tpu_sc as plsc

_LAMBDA = 0.2
_E = 160000
_D = 256
_NW = 32
_K = 128
_EPW0 = 6400
_EPW1 = 3840
_EPS = _EPW0 + _EPW1
_NPAIR0 = _EPW0 // (2 * _K)
_NPAIR1 = _EPW1 // (2 * _K)
_E_PAD = 16 * _EPS


def _rsqrt(p):
    i = lax.bitcast_convert_type(p, jnp.int32)
    i = jnp.int32(0x5F3759DF) - (i >> 1)
    y = lax.bitcast_convert_type(i, jnp.float32)
    y = y * (1.5 - 0.5 * p * y * y)
    y = y * (1.5 - 0.5 * p * y * y)
    y = y * (1.5 - 0.5 * p * y * y)
    return y


def _build():
    mesh = plsc.VectorSubcoreMesh(core_axis_name="c", subcore_axis_name="s")

    @functools.partial(
        pl.kernel,
        out_type=jax.ShapeDtypeStruct((_NW, 32), jnp.float32),
        mesh=mesh,
        compiler_params=pltpu.CompilerParams(needs_layout_passes=False),
        scratch_types=[
            pltpu.VMEM((_K,), jnp.int32),
            pltpu.VMEM((_K,), jnp.int32),
            pltpu.VMEM((_K, _D // 2), jnp.float32),
            pltpu.VMEM((_K, _D // 2), jnp.float32),
            pltpu.VMEM((_K,), jnp.int32),
            pltpu.VMEM((_K,), jnp.int32),
            pltpu.VMEM((_K, _D // 2), jnp.float32),
            pltpu.VMEM((_K, _D // 2), jnp.float32),
            pltpu.VMEM((32,), jnp.float32),
            pltpu.SemaphoreType.DMA,
            pltpu.SemaphoreType.DMA,
            pltpu.SemaphoreType.DMA,
            pltpu.SemaphoreType.DMA,
        ],
    )
    def k(table, sidx_hbm, tidx_hbm, out,
          sidx0, tidx0, srows0, trows0,
          sidx1, tidx1, srows1, trows1,
          outbuf, ss0, st0, ss1, st1):
        cid = lax.axis_index("c")
        sid = lax.axis_index("s")
        wid = sid * 2 + cid
        base_w = sid * _EPS + cid * _EPW0
        npair = jnp.where(cid == 0, _NPAIR0, _NPAIR1)
        bufs = ((sidx0, tidx0, srows0, trows0, ss0, st0),
                (sidx1, tidx1, srows1, trows1, ss1, st1))

        def fire(jb, b):
            sidx, tidx, srows, trows, ss, st = bufs[b]
            base = base_w + jb * _K
            pltpu.sync_copy(sidx_hbm.at[pl.ds(base, _K)], sidx)
            pltpu.sync_copy(tidx_hbm.at[pl.ds(base, _K)], tidx)
            pltpu.async_copy(table.at[sidx], srows, ss)
            pltpu.async_copy(table.at[tidx], trows, st)

        def drain(b):
            sidx, tidx, srows, trows, ss, st = bufs[b]
            pltpu.make_async_copy(table.at[sidx], srows, ss).wait()
            pltpu.make_async_copy(table.at[tidx], trows, st).wait()

        def compute(jb, b, acc):
            _, _, srows, trows, _, _ = bufs[b]
            base = base_w + jb * _K

            def edge_body(i, carry):
                a_cos, a_sq = carry
                d = ns = nt = None
                for c in range(8):
                    sa, sb = plsc.unpack(
                        plsc.bitcast(srows[i, pl.ds(c * 16, 16)], jnp.bfloat16),
                        format=plsc.PackFormat.INTERLEAVED,
                        preferred_element_type=jnp.float32)
                    ta, tb = plsc.unpack(
                        plsc.bitcast(trows[i, pl.ds(c * 16, 16)], jnp.bfloat16),
                        format=plsc.PackFormat.INTERLEAVED,
                        preferred_element_type=jnp.float32)
                    if d is None:
                        d = sa * ta + sb * tb
                        ns = sa * sa + sb * sb
                        nt = ta * ta + tb * tb
                    else:
                        d = d + sa * ta + sb * tb
                        ns = ns + sa * sa + sb * sb
                        nt = nt + ta * ta + tb * tb
                dsc = jnp.sum(d)
                nssc = jnp.sum(ns)
                ntsc = jnp.sum(nt)
                p = jnp.maximum(nssc, 1e-16) * jnp.maximum(ntsc, 1e-16)
                cos = dsc * _rsqrt(p)
                valid = (base + i) < _E
                a_cos = a_cos + jnp.where(valid, 1.0 - cos, 0.0)
                a_sq = a_sq + jnp.where(valid, nssc + ntsc - 2.0 * dsc, 0.0)
                return (a_cos, a_sq)

            return plsc.parallel_loop(0, _K, 1, unroll=4, carry=acc)(edge_body)

        fire(0, 0)

        def pair_body(j2, acc):
            a = 2 * j2
            fire(a + 1, 1)
            drain(0)
            acc = compute(a, 0, acc)

            @pl.when(j2 < npair - 1)
            def _():
                fire(a + 2, 0)

            drain(1)
            acc = compute(a + 1, 1, acc)
            return acc

        acc_cos, acc_sq = lax.fori_loop(
            0, npair, pair_body, (jnp.float32(0.0), jnp.float32(0.0)))
        lanes = lax.iota(jnp.int32, 16)
        first = lanes < 1
        outbuf[pl.ds(0, 16)] = jnp.where(
            first, acc_cos * (1.0 / _E), 0.0)
        outbuf[pl.ds(16, 16)] = jnp.where(
            first, acc_sq * (_LAMBDA / (_E * _D)), 0.0)
        pltpu.sync_copy(outbuf, out.at[wid])

    return k


_sc_kernel = _build()


def kernel(embedding, edge_index):
    ei = edge_index.astype(jnp.int32)
    pad = _E_PAD - _E
    src = jnp.concatenate([ei[0], jnp.zeros((pad,), jnp.int32)])
    tgt = jnp.concatenate([ei[1], jnp.zeros((pad,), jnp.int32)])
    lo = lax.bitcast_convert_type(
        embedding[:, :_D // 2].astype(jnp.bfloat16), jnp.uint16)
    hi = lax.bitcast_convert_type(
        embedding[:, _D // 2:].astype(jnp.bfloat16), jnp.uint16)
    table = lax.bitcast_convert_type(
        lo.astype(jnp.uint32) | (hi.astype(jnp.uint32) << 16), jnp.float32)
    parts = _sc_kernel(table, src, tgt)
    return jnp.sum(parts)
